# baseline (speedup 1.0000x reference)
import functools

import jax
import jax.numpy as jnp
from jax import lax
from jax.experimental import pallas as pl
from jax.experimental.pallas import tpu as pltpu
from jax.experimental.pallas import tpu_sc as plsc

_K = 10
_NEG = -1.0
_L = 16


def _nms_rowmax_body(h_ref, rmax_ref, vpad_ref, hpad_ref):
    _, H, W = h_ref.shape
    TOP, LP = 8, 128
    h = h_ref[0]
    ninf = jnp.float32(-jnp.inf)

    @pl.when(pl.program_id(0) == 0)
    def _():
        vpad_ref[pl.ds(0, TOP), :] = jnp.full((TOP, W), ninf, jnp.float32)
        vpad_ref[pl.ds(TOP + H, TOP), :] = jnp.full((TOP, W), ninf,
                                                    jnp.float32)
        hpad_ref[:, pl.ds(0, LP)] = jnp.full((H, LP), ninf, jnp.float32)
        hpad_ref[:, pl.ds(LP + W, LP)] = jnp.full((H, LP), ninf,
                                                  jnp.float32)

    vpad_ref[pl.ds(TOP, H), :] = h
    v = vpad_ref[pl.ds(TOP - 2, H), :]
    for s in range(1, 5):
        v = jnp.maximum(v, vpad_ref[pl.ds(TOP - 2 + s, H), :])

    hpad_ref[:, pl.ds(LP, W)] = v
    m5 = hpad_ref[:, pl.ds(LP - 2, W)]
    for s in range(1, 5):
        m5 = jnp.maximum(m5, hpad_ref[:, pl.ds(LP - 2 + s, W)])

    nms = jnp.where(m5 == h, h, 0.0)
    rmax_ref[0] = jnp.max(nms, axis=1, keepdims=True)


def _sc_topk_body(h_hbm, rm_hbm, val_o, x_o, y_o,
                  rm_v, slab_v, vm_v, cache_v, rbmax_v, bmax_v,
                  sv_v, sx_v, sy_v, sem0, sem1, sem2, sem3, *,
                  nv, h, w, img_off):
    sems = [sem0, sem1, sem2, sem3]
    NV, H, W = nv, h, w
    SR = 16
    NB = W // _L
    NBB = H // _L
    wid = lax.axis_index("s") * 2 + lax.axis_index("c")

    @pl.when(wid < NV)
    def _():
        img = wid
        img_f = wid + img_off
        pltpu.sync_copy(rm_hbm.at[img], rm_v)
        lanes = lax.iota(jnp.int32, _L)
        lane0 = lanes == 0

        def splat_f(x):
            return jnp.full((_L,), x, jnp.float32)

        def splat_i(x):
            return jnp.full((_L,), x, jnp.int32)

        zero_i = splat_i(0)
        big = splat_i(1 << 20)

        for half in range(2):
            acc = splat_f(-3e38)
            base = (lanes + half * _L) * _L
            for j in range(_L):
                acc = jnp.maximum(
                    acc, plsc.load_gather(rm_v, [base + j, zero_i]))
            plsc.store_scatter(bmax_v, [lanes + half * _L], acc)

        def bmax_argmax():
            v0 = plsc.load_gather(bmax_v, [lanes])
            v1 = plsc.load_gather(bmax_v, [lanes + _L])
            m = jnp.max(jnp.maximum(v0, v1))
            m_s = splat_f(m)
            c0 = jnp.min(jnp.where(v0 == m_s, lanes, big))
            c1 = jnp.min(jnp.where(v1 == m_s, lanes + _L, big))
            blk = jnp.minimum(c0, c1)
            rows = splat_i(blk * _L) + lanes
            rv = plsc.load_gather(rm_v, [rows, zero_i])
            r = jnp.min(jnp.where(rv == m_s, rows, big))
            return m, m_s, blk, r

        def refresh_bmax(blk):
            rows = splat_i(blk * _L) + lanes
            nb = jnp.max(plsc.load_gather(rm_v, [rows, zero_i]))
            plsc.store_scatter(bmax_v, [splat_i(blk)], splat_f(nb),
                               mask=lane0)

        rsel, msel, bsel = [], [], []
        for k in range(_K):
            m, m_s, blk, r = bmax_argmax()
            rsel.append(r)
            msel.append(m)
            bsel.append(blk)
            plsc.store_scatter(rm_v, [splat_i(r), zero_i], splat_f(-1.0),
                               mask=lane0)
            refresh_bmax(blk)

        rows_vec = splat_i(-1)
        mval_vec = splat_f(0.0)
        for k in range(_K):
            sel = lanes == k
            rows_vec = jnp.where(sel, splat_i(rsel[k]), rows_vec)
            mval_vec = jnp.where(sel, splat_f(msel[k]), mval_vec)
        plsc.store_scatter(rm_v, [rows_vec, zero_i], mval_vec,
                           mask=lanes < _K)
        for k in range(_K):
            refresh_bmax(bsel[k])

        NRING = 4
        for k in range(_K):
            bsel[k] = pl.multiple_of(
                jnp.minimum(jnp.maximum(rsel[k] - 2, 0) // 8 * 8, H - SR), 8)
        copies = [
            pltpu.async_copy(h_hbm.at[img_f, pl.ds(bsel[k], SR)],
                             slab_v.at[k], sems[k])
            for k in range(NRING)]

        for k in range(_K):
            copies[k].wait()
            r, b = rsel[k], bsel[k]
            rels = [splat_i(jnp.clip(r + d - 2, 0, H - 1) - b)
                    for d in range(5)]
            c_rel = splat_i(r - b)
            k_s = splat_i(k % NRING)
            ck_s = splat_i(k)

            def vpass(i, _, rels=rels, k_s=k_s):
                cols = splat_i(i * _L) + lanes
                vm = plsc.load_gather(slab_v, [k_s, rels[0], cols])
                for d in range(1, 5):
                    vm = jnp.maximum(
                        vm, plsc.load_gather(slab_v, [k_s, rels[d], cols]))
                plsc.store_scatter(vm_v, [cols], vm)
                return 0

            lax.fori_loop(0, NB, vpass, 0)

            def hpass(i, _, c_rel=c_rel, k_s=k_s, ck_s=ck_s):
                cols = splat_i(i * _L) + lanes
                hm = splat_f(-3e38)
                for s in range(-2, 3):
                    sc = jnp.clip(cols + s, 0, W - 1)
                    hm = jnp.maximum(hm, plsc.load_gather(vm_v, [sc]))
                center = plsc.load_gather(slab_v, [k_s, c_rel, cols])
                nmsrow = jnp.where(hm == center, center, 0.0)
                plsc.store_scatter(cache_v, [ck_s, cols], nmsrow)
                plsc.store_scatter(rbmax_v, [ck_s, splat_i(i)],
                                   splat_f(jnp.max(nmsrow)), mask=lane0)
                return 0

            lax.fori_loop(0, NB, hpass, 0)

            if k + NRING < _K:
                copies.append(
                    pltpu.async_copy(
                        h_hbm.at[img_f, pl.ds(bsel[k + NRING], SR)],
                        slab_v.at[(k + NRING) % NRING],
                        sems[(k + NRING) % NRING]))

        vals = splat_f(0.0)
        xs = splat_i(0)
        ys = splat_i(0)
        for k in range(_K):
            m, m_s, blk, r = bmax_argmax()
            r_s = splat_i(r)
            slot = jnp.min(jnp.where(rows_vec == r_s, lanes, big))
            slot_s = splat_i(slot)

            u0 = plsc.load_gather(rbmax_v, [slot_s, lanes])
            u1 = plsc.load_gather(rbmax_v, [slot_s, lanes + _L])
            j0 = jnp.min(jnp.where(u0 == m_s, lanes, big))
            j1 = jnp.min(jnp.where(u1 == m_s, lanes + _L, big))
            jc = jnp.minimum(j0, j1)
            cols = splat_i(jc * _L) + lanes
            cv = plsc.load_gather(cache_v, [slot_s, cols])
            c = jnp.min(jnp.where(cv == m_s, cols, big))
            c_s = splat_i(c)

            plsc.store_scatter(cache_v, [slot_s, c_s], splat_f(_NEG),
                               mask=lane0)
            nb = jnp.max(plsc.load_gather(cache_v, [slot_s, cols]))
            plsc.store_scatter(rbmax_v, [slot_s, splat_i(jc)], splat_f(nb),
                               mask=lane0)
            w0 = plsc.load_gather(rbmax_v, [slot_s, lanes])
            w1 = plsc.load_gather(rbmax_v, [slot_s, lanes + _L])
            newmax = jnp.max(jnp.maximum(w0, w1))
            plsc.store_scatter(rm_v, [r_s, zero_i], splat_f(newmax),
                               mask=lane0)
            refresh_bmax(blk)

            sel = lanes == k
            vals = jnp.where(sel, m_s, vals)
            xs = jnp.where(sel, c_s, xs)
            ys = jnp.where(sel, r_s, ys)

        sv_v[...] = vals
        sx_v[...] = xs
        sy_v[...] = ys
        obase = pl.multiple_of(img * _L, 8)
        pltpu.sync_copy(sv_v, val_o.at[pl.ds(obase, _L)])
        pltpu.sync_copy(sx_v, x_o.at[pl.ds(obase, _L)])
        pltpu.sync_copy(sy_v, y_o.at[pl.ds(obase, _L)])


def kernel(heatmaps):
    N, V, H, W = heatmaps.shape
    NV = N * V
    h = heatmaps.reshape(NV, H, W)

    rmax = pl.pallas_call(
        _nms_rowmax_body,
        grid=(NV,),
        in_specs=[pl.BlockSpec((1, H, W), lambda i: (i, 0, 0))],
        out_specs=pl.BlockSpec((1, H, 1), lambda i: (i, 0, 0)),
        out_shape=jax.ShapeDtypeStruct((NV, H, 1), jnp.float32),
        scratch_shapes=[pltpu.VMEM((H + 16, W), jnp.float32),
                        pltpu.VMEM((H, W + 256), jnp.float32)],
    )(h)

    mesh = plsc.VectorSubcoreMesh(core_axis_name="c", subcore_axis_name="s",
                                  num_cores=2, num_subcores=16)
    sc = functools.partial(
        pl.kernel,
        out_type=[jax.ShapeDtypeStruct((NV * _L,), jnp.float32),
                  jax.ShapeDtypeStruct((NV * _L,), jnp.int32),
                  jax.ShapeDtypeStruct((NV * _L,), jnp.int32)],
        mesh=mesh,
        compiler_params=pltpu.CompilerParams(needs_layout_passes=False),
        scratch_types=[pltpu.VMEM((H, 1), jnp.float32),
                       pltpu.VMEM((4, 16, W), jnp.float32),
                       pltpu.VMEM((W,), jnp.float32),
                       pltpu.VMEM((_L, W), jnp.float32),
                       pltpu.VMEM((_L, 2 * _L), jnp.float32),
                       pltpu.VMEM((2 * _L,), jnp.float32),
                       pltpu.VMEM((_L,), jnp.float32),
                       pltpu.VMEM((_L,), jnp.int32),
                       pltpu.VMEM((_L,), jnp.int32)]
                      + [pltpu.SemaphoreType.DMA] * 4,
    )(functools.partial(_sc_topk_body, nv=NV, h=H, w=W, img_off=0))
    val_p, x_p, y_p = sc(h, rmax)

    val_k = val_p.reshape(NV, _L)[:, :_K].reshape(N, V, _K)
    ind_k = jnp.stack([x_p.reshape(NV, _L)[:, :_K],
                       y_p.reshape(NV, _L)[:, :_K]],
                      axis=-1).reshape(N, V, _K, 2)
    return ind_k, val_k


# speedup vs baseline: 1.2212x; 1.2212x over previous
import functools

import jax
import jax.numpy as jnp
from jax import lax
from jax.experimental import pallas as pl
from jax.experimental.pallas import tpu as pltpu
from jax.experimental.pallas import tpu_sc as plsc

_K = 10
_NEG = -1.0
_L = 16


def _nms_rowmax_body(h_ref, rmax_ref):
    _, H, W = h_ref.shape
    h = h_ref[0]
    ninf = jnp.float32(-jnp.inf)

    def sh_up(a, s):
        return jnp.concatenate([a[s:], jnp.full((s, W), ninf, a.dtype)], axis=0)

    def sh_dn(a, s):
        return jnp.concatenate([jnp.full((s, W), ninf, a.dtype), a[: H - s]], axis=0)

    v = jnp.maximum(h, jnp.maximum(sh_up(h, 1), sh_up(h, 2)))
    v = jnp.maximum(v, jnp.maximum(sh_dn(h, 1), sh_dn(h, 2)))

    def sh_l(a, s):
        return jnp.concatenate([a[:, s:], jnp.full((H, s), ninf, a.dtype)], axis=1)

    def sh_r(a, s):
        return jnp.concatenate([jnp.full((H, s), ninf, a.dtype), a[:, : W - s]], axis=1)

    m5 = jnp.maximum(v, jnp.maximum(sh_l(v, 1), sh_l(v, 2)))
    m5 = jnp.maximum(m5, jnp.maximum(sh_r(v, 1), sh_r(v, 2)))

    nms = jnp.where(m5 == h, h, 0.0)
    rmax_ref[0] = jnp.max(nms, axis=1, keepdims=True)


def _sc_topk_body(h_hbm, rm_hbm, val_o, x_o, y_o,
                  rm_v, slab_v, vm_v, cache_v, rbmax_v, bmax_v,
                  sv_v, sx_v, sy_v, sem0, sem1, sem2, sem3, *,
                  nv, h, w, img_off):
    sems = [sem0, sem1, sem2, sem3]
    NV, H, W = nv, h, w
    SR = 16
    NB = W // _L
    NBB = H // _L
    wid = lax.axis_index("s") * 2 + lax.axis_index("c")

    @pl.when(wid < NV)
    def _():
        img = wid
        img_f = wid + img_off
        pltpu.sync_copy(rm_hbm.at[img], rm_v)
        lanes = lax.iota(jnp.int32, _L)
        lane0 = lanes == 0

        def splat_f(x):
            return jnp.full((_L,), x, jnp.float32)

        def splat_i(x):
            return jnp.full((_L,), x, jnp.int32)

        zero_i = splat_i(0)
        big = splat_i(1 << 20)

        for half in range(2):
            acc = splat_f(-3e38)
            base = (lanes + half * _L) * _L
            for j in range(_L):
                acc = jnp.maximum(
                    acc, plsc.load_gather(rm_v, [base + j, zero_i]))
            plsc.store_scatter(bmax_v, [lanes + half * _L], acc)

        def bmax_argmax():
            v0 = plsc.load_gather(bmax_v, [lanes])
            v1 = plsc.load_gather(bmax_v, [lanes + _L])
            m = jnp.max(jnp.maximum(v0, v1))
            m_s = splat_f(m)
            c0 = jnp.min(jnp.where(v0 == m_s, lanes, big))
            c1 = jnp.min(jnp.where(v1 == m_s, lanes + _L, big))
            blk = jnp.minimum(c0, c1)
            rows = splat_i(blk * _L) + lanes
            rv = plsc.load_gather(rm_v, [rows, zero_i])
            r = jnp.min(jnp.where(rv == m_s, rows, big))
            return m, m_s, blk, r

        def refresh_bmax(blk):
            rows = splat_i(blk * _L) + lanes
            nb = jnp.max(plsc.load_gather(rm_v, [rows, zero_i]))
            plsc.store_scatter(bmax_v, [splat_i(blk)], splat_f(nb),
                               mask=lane0)

        rsel, msel, bsel = [], [], []
        for k in range(_K):
            m, m_s, blk, r = bmax_argmax()
            rsel.append(r)
            msel.append(m)
            bsel.append(blk)
            plsc.store_scatter(rm_v, [splat_i(r), zero_i], splat_f(-1.0),
                               mask=lane0)
            refresh_bmax(blk)

        rows_vec = splat_i(-1)
        mval_vec = splat_f(0.0)
        for k in range(_K):
            sel = lanes == k
            rows_vec = jnp.where(sel, splat_i(rsel[k]), rows_vec)
            mval_vec = jnp.where(sel, splat_f(msel[k]), mval_vec)
        plsc.store_scatter(rm_v, [rows_vec, zero_i], mval_vec,
                           mask=lanes < _K)
        for k in range(_K):
            refresh_bmax(bsel[k])

        NRING = 4
        for k in range(_K):
            bsel[k] = pl.multiple_of(
                jnp.minimum(jnp.maximum(rsel[k] - 2, 0) // 8 * 8, H - SR), 8)
        copies = [
            pltpu.async_copy(h_hbm.at[img_f, pl.ds(bsel[k], SR)],
                             slab_v.at[k], sems[k])
            for k in range(NRING)]

        for k in range(_K):
            copies[k].wait()
            r, b = rsel[k], bsel[k]
            rels = [splat_i(jnp.clip(r + d - 2, 0, H - 1) - b)
                    for d in range(5)]
            c_rel = splat_i(r - b)
            k_s = splat_i(k % NRING)
            ck_s = splat_i(k)

            def vpass(i, _, rels=rels, k_s=k_s):
                cols = splat_i(i * _L) + lanes
                vm = plsc.load_gather(slab_v, [k_s, rels[0], cols])
                for d in range(1, 5):
                    vm = jnp.maximum(
                        vm, plsc.load_gather(slab_v, [k_s, rels[d], cols]))
                plsc.store_scatter(vm_v, [cols], vm)
                return 0

            lax.fori_loop(0, NB, vpass, 0)

            def hpass(i, _, c_rel=c_rel, k_s=k_s, ck_s=ck_s):
                cols = splat_i(i * _L) + lanes
                hm = splat_f(-3e38)
                for s in range(-2, 3):
                    sc = jnp.clip(cols + s, 0, W - 1)
                    hm = jnp.maximum(hm, plsc.load_gather(vm_v, [sc]))
                center = plsc.load_gather(slab_v, [k_s, c_rel, cols])
                nmsrow = jnp.where(hm == center, center, 0.0)
                plsc.store_scatter(cache_v, [ck_s, cols], nmsrow)
                plsc.store_scatter(rbmax_v, [ck_s, splat_i(i)],
                                   splat_f(jnp.max(nmsrow)), mask=lane0)
                return 0

            lax.fori_loop(0, NB, hpass, 0)

            if k + NRING < _K:
                copies.append(
                    pltpu.async_copy(
                        h_hbm.at[img_f, pl.ds(bsel[k + NRING], SR)],
                        slab_v.at[(k + NRING) % NRING],
                        sems[(k + NRING) % NRING]))

        vals = splat_f(0.0)
        xs = splat_i(0)
        ys = splat_i(0)
        for k in range(_K):
            m, m_s, blk, r = bmax_argmax()
            r_s = splat_i(r)
            slot = jnp.min(jnp.where(rows_vec == r_s, lanes, big))
            slot_s = splat_i(slot)

            u0 = plsc.load_gather(rbmax_v, [slot_s, lanes])
            u1 = plsc.load_gather(rbmax_v, [slot_s, lanes + _L])
            j0 = jnp.min(jnp.where(u0 == m_s, lanes, big))
            j1 = jnp.min(jnp.where(u1 == m_s, lanes + _L, big))
            jc = jnp.minimum(j0, j1)
            cols = splat_i(jc * _L) + lanes
            cv = plsc.load_gather(cache_v, [slot_s, cols])
            c = jnp.min(jnp.where(cv == m_s, cols, big))
            c_s = splat_i(c)

            plsc.store_scatter(cache_v, [slot_s, c_s], splat_f(_NEG),
                               mask=lane0)
            nb = jnp.max(plsc.load_gather(cache_v, [slot_s, cols]))
            plsc.store_scatter(rbmax_v, [slot_s, splat_i(jc)], splat_f(nb),
                               mask=lane0)
            w0 = plsc.load_gather(rbmax_v, [slot_s, lanes])
            w1 = plsc.load_gather(rbmax_v, [slot_s, lanes + _L])
            newmax = jnp.max(jnp.maximum(w0, w1))
            plsc.store_scatter(rm_v, [r_s, zero_i], splat_f(newmax),
                               mask=lane0)
            refresh_bmax(blk)

            sel = lanes == k
            vals = jnp.where(sel, m_s, vals)
            xs = jnp.where(sel, c_s, xs)
            ys = jnp.where(sel, r_s, ys)

        sv_v[...] = vals
        sx_v[...] = xs
        sy_v[...] = ys
        obase = pl.multiple_of(img * _L, 8)
        pltpu.sync_copy(sv_v, val_o.at[pl.ds(obase, _L)])
        pltpu.sync_copy(sx_v, x_o.at[pl.ds(obase, _L)])
        pltpu.sync_copy(sy_v, y_o.at[pl.ds(obase, _L)])


def kernel(heatmaps):
    N, V, H, W = heatmaps.shape
    NV = N * V
    h = heatmaps.reshape(NV, H, W)

    rmax = pl.pallas_call(
        _nms_rowmax_body,
        grid=(NV,),
        in_specs=[pl.BlockSpec((1, H, W), lambda i: (i, 0, 0))],
        out_specs=pl.BlockSpec((1, H, 1), lambda i: (i, 0, 0)),
        out_shape=jax.ShapeDtypeStruct((NV, H, 1), jnp.float32),
    )(h)

    mesh = plsc.VectorSubcoreMesh(core_axis_name="c", subcore_axis_name="s",
                                  num_cores=2, num_subcores=16)
    sc = functools.partial(
        pl.kernel,
        out_type=[jax.ShapeDtypeStruct((NV * _L,), jnp.float32),
                  jax.ShapeDtypeStruct((NV * _L,), jnp.int32),
                  jax.ShapeDtypeStruct((NV * _L,), jnp.int32)],
        mesh=mesh,
        compiler_params=pltpu.CompilerParams(needs_layout_passes=False),
        scratch_types=[pltpu.VMEM((H, 1), jnp.float32),
                       pltpu.VMEM((4, 16, W), jnp.float32),
                       pltpu.VMEM((W,), jnp.float32),
                       pltpu.VMEM((_L, W), jnp.float32),
                       pltpu.VMEM((_L, 2 * _L), jnp.float32),
                       pltpu.VMEM((2 * _L,), jnp.float32),
                       pltpu.VMEM((_L,), jnp.float32),
                       pltpu.VMEM((_L,), jnp.int32),
                       pltpu.VMEM((_L,), jnp.int32)]
                      + [pltpu.SemaphoreType.DMA] * 4,
    )(functools.partial(_sc_topk_body, nv=NV, h=H, w=W, img_off=0))
    val_p, x_p, y_p = sc(h, rmax)

    val_k = val_p.reshape(NV, _L)[:, :_K].reshape(N, V, _K)
    ind_k = jnp.stack([x_p.reshape(NV, _L)[:, :_K],
                       y_p.reshape(NV, _L)[:, :_K]],
                      axis=-1).reshape(N, V, _K, 2)
    return ind_k, val_k
